# baseline (speedup 1.0000x reference)
import jax
import jax.numpy as jnp
from jax import lax
from jax.experimental import pallas as pl
from jax.experimental.pallas import tpu as pltpu
from jax.experimental.pallas import tpu_sc as plsc

_N = 10000
_D = 128
_DH = _D // 2
_NC = 2
_NS = 16
_L = 16

_C = 128
_E = 320000
_EPAD = ((_E + _NS * _C - 1) // (_NS * _C)) * (_NS * _C)
_NCHUNK = _EPAD // (_NS * _C)
_NBUF = 3

_RPT = 640
_RPT_LAST = _N - 15 * _RPT
_ZB = 80


def _spmm_body(x2_hbm, src_hbm, dst_hbm, w_hbm, out_hbm,
               src_v, dst_v, w_v,
               rows0, rows1, rows2, acc_sh,
               sem0, sem1, sem2, ssem0, ssem1, ssem2):
    c = lax.axis_index("c")
    s = lax.axis_index("s")
    rows = (rows0, rows1, rows2)
    sems = (sem0, sem1, sem2)
    ssems = (ssem0, ssem1, ssem2)

    pltpu.sync_copy(src_hbm.at[c, s], src_v)
    pltpu.sync_copy(dst_hbm.at[s], dst_v)
    pltpu.sync_copy(w_hbm.at[s], w_v)

    zero16 = jnp.zeros((_L,), jnp.float32)

    def zrow(i, _):
        for j in range(_DH // _L):
            rows1[i, pl.ds(j * _L, _L)] = zero16
        return 0
    lax.fori_loop(0, _ZB, zrow, 0)

    r0 = s * _RPT
    nz = jnp.where(s < _NS - 1, _RPT // _ZB, _RPT_LAST // _ZB)

    def zcopy(i, _):
        off = pl.multiple_of(r0 + i * _ZB, 8)
        pltpu.sync_copy(rows1.at[pl.ds(0, _ZB)], acc_sh.at[pl.ds(off, _ZB)])
        return 0
    lax.fori_loop(0, nz, zcopy, 0)

    for b in range(_NBUF):
        pltpu.async_copy(x2_hbm.at[src_v.at[b]], rows[b], sems[b])

    plsc.subcore_barrier()

    def drain_scatter(b):
        pltpu.make_async_copy(x2_hbm.at[pl.ds(0, _C)], rows[b],
                              ssems[b]).wait()

    def process(kb, b):
        pltpu.make_async_copy(x2_hbm.at[src_v.at[kb]], rows[b], sems[b]).wait()

        @plsc.parallel_loop(0, _C // _L, unroll=8)
        def mul(g):
            wv = w_v[kb, pl.ds(g * _L, _L)]
            for l in range(_L):
                w = wv[l]
                e = g * _L + l
                for j in range(_DH // _L):
                    sl = pl.ds(j * _L, _L)
                    rows[b][e, sl] = rows[b][e, sl] * w

        pltpu.async_copy(rows[b], acc_sh.at[dst_v.at[kb]], ssems[b], add=True)

    def body(i, _):
        k = i * _NBUF
        for b in range(_NBUF):
            kb = k + b
            process(kb, b)

            bp = (b + _NBUF - 1) % _NBUF

            @pl.when(jnp.logical_and(kb >= 1,
                                     kb - 1 + _NBUF <= _NCHUNK - 1))
            def _():
                drain_scatter(bp)
                pltpu.async_copy(x2_hbm.at[src_v.at[kb - 1 + _NBUF]],
                                 rows[bp], sems[bp])
        return 0
    lax.fori_loop(0, (_NCHUNK - 1) // _NBUF, body, 0)

    process(_NCHUNK - 1, 0)

    for b in range(_NBUF):
        drain_scatter(b)

    plsc.subcore_barrier()
    ro = pl.multiple_of(r0, 8)

    @pl.when(s < _NS - 1)
    def _():
        pltpu.sync_copy(acc_sh.at[pl.ds(ro, _RPT)],
                        out_hbm.at[pl.ds(ro, _RPT), c])

    @pl.when(s == _NS - 1)
    def _():
        pltpu.sync_copy(acc_sh.at[pl.ds(ro, _RPT_LAST)],
                        out_hbm.at[pl.ds(ro, _RPT_LAST), c])


def _spmm(x2, src4, dst3, w3):
    mesh = plsc.VectorSubcoreMesh(core_axis_name="c", subcore_axis_name="s")
    f = pl.kernel(
        _spmm_body,
        mesh=mesh,
        compiler_params=pltpu.CompilerParams(use_tc_tiling_on_sc=False),
        out_type=jax.ShapeDtypeStruct((_N, _NC, _DH), jnp.float32),
        scratch_types=[
            pltpu.VMEM((_NCHUNK, _C), jnp.int32),
            pltpu.VMEM((_NCHUNK, _C), jnp.int32),
            pltpu.VMEM((_NCHUNK, _C), jnp.float32),
            pltpu.VMEM((_C, _DH), jnp.float32),
            pltpu.VMEM((_C, _DH), jnp.float32),
            pltpu.VMEM((_C, _DH), jnp.float32),
            pltpu.VMEM_SHARED((_N, _DH), jnp.float32),
            pltpu.SemaphoreType.DMA,
            pltpu.SemaphoreType.DMA,
            pltpu.SemaphoreType.DMA,
            pltpu.SemaphoreType.DMA,
            pltpu.SemaphoreType.DMA,
            pltpu.SemaphoreType.DMA,
        ],
    )
    return f(x2, src4, dst3, w3)


_R = 2000


def _tc1_body(x_ref, p_ref, wa_ref, ba_ref, wla_ref, wlb_ref, bl_ref, h_ref):
    agg = jnp.maximum(
        jnp.dot(p_ref[...], wa_ref[...], preferred_element_type=jnp.float32)
        + ba_ref[...], 0.0)
    hv = (jnp.dot(x_ref[...], wla_ref[...], preferred_element_type=jnp.float32)
          + jnp.dot(agg, wlb_ref[...], preferred_element_type=jnp.float32)
          + bl_ref[...])
    h_ref[...] = jnp.maximum(hv, 0.0)


def _tc2_body(h_ref, q_ref, wa_ref, ba_ref, wla_ref, wlb_ref, bl_ref, o_ref):
    agg = jnp.maximum(
        jnp.dot(q_ref[...], wa_ref[...], preferred_element_type=jnp.float32)
        + ba_ref[...], 0.0)
    ov = (jnp.dot(h_ref[...], wla_ref[...], preferred_element_type=jnp.float32)
          + jnp.dot(agg, wlb_ref[...], preferred_element_type=jnp.float32)
          + bl_ref[...])
    nrm = jnp.sqrt(jnp.sum(ov * ov, axis=1, keepdims=True))
    o_ref[...] = ov / jnp.maximum(nrm, 1e-12)


def _dense_layer(body, xh, p, Wa, ba, Wl, bl):
    grid = (_N // _R,)
    specs = [
        pl.BlockSpec((_R, _D), lambda i: (i, 0)),
        pl.BlockSpec((_R, _D), lambda i: (i, 0)),
        pl.BlockSpec((_D, _D), lambda i: (0, 0)),
        pl.BlockSpec((1, _D), lambda i: (0, 0)),
        pl.BlockSpec((_D, _D), lambda i: (0, 0)),
        pl.BlockSpec((_D, _D), lambda i: (0, 0)),
        pl.BlockSpec((1, _D), lambda i: (0, 0)),
    ]
    return pl.pallas_call(
        body,
        grid=grid,
        in_specs=specs,
        out_specs=pl.BlockSpec((_R, _D), lambda i: (i, 0)),
        out_shape=jax.ShapeDtypeStruct((_N, _D), jnp.float32),
    )(xh, p, Wa, ba.reshape(1, _D), Wl[:_D], Wl[_D:], bl.reshape(1, _D))


def kernel(x, edge_index, edge_weight, Wa0, ba0, Wa1, ba1, Wl0, bl0, Wl1, bl1):
    npad = _EPAD - _E
    src = edge_index[1].astype(jnp.int32)
    dst = jnp.pad(edge_index[0].astype(jnp.int32), (0, npad))
    w = jnp.pad(edge_weight.astype(jnp.float32), (0, npad))

    src2 = src * 2
    src4 = jnp.stack([src2, src2 + 1])
    src4 = jnp.pad(src4, ((0, 0), (0, npad)))
    src4 = src4.reshape(_NC, _NS, _NCHUNK, _C)
    dst3 = dst.reshape(_NS, _NCHUNK, _C)
    w3 = w.reshape(_NS, _NCHUNK, _C)

    x2 = x.reshape(2 * _N, _DH)
    p = _spmm(x2, src4, dst3, w3).reshape(_N, _D)
    h = _dense_layer(_tc1_body, x, p, Wa0, ba0, Wl0, bl0)
    q = _spmm(h.reshape(2 * _N, _DH), src4, dst3, w3).reshape(_N, _D)
    return _dense_layer(_tc2_body, h, q, Wa1, ba1, Wl1, bl1)


# speedup vs baseline: 1.0938x; 1.0938x over previous
import jax
import jax.numpy as jnp
from jax import lax
from jax.experimental import pallas as pl
from jax.experimental.pallas import tpu as pltpu
from jax.experimental.pallas import tpu_sc as plsc

_N = 10000
_D = 128
_DH = _D // 2
_NC = 2
_NS = 16
_L = 16

_C = 128
_E = 320000
_EPAD = ((_E + _NS * _C - 1) // (_NS * _C)) * (_NS * _C)
_NCHUNK = _EPAD // (_NS * _C)
_NBUF = 3

_RPT = 640
_RPT_LAST = _N - 15 * _RPT
_ZB = 80


def _spmm_body(x2_hbm, src_hbm, dst_hbm, w_hbm, out_hbm,
               src_v, dst_v, w_v,
               rows0, rows1, rows2, acc_sh,
               sem0, sem1, sem2):
    c = lax.axis_index("c")
    s = lax.axis_index("s")
    rows = (rows0, rows1, rows2)
    sems = (sem0, sem1, sem2)

    pltpu.sync_copy(src_hbm.at[c, s], src_v)
    pltpu.sync_copy(dst_hbm.at[s], dst_v)
    pltpu.sync_copy(w_hbm.at[s], w_v)

    zero16 = jnp.zeros((_L,), jnp.float32)

    def zrow(i, _):
        for j in range(_DH // _L):
            rows1[i, pl.ds(j * _L, _L)] = zero16
        return 0
    lax.fori_loop(0, _ZB, zrow, 0)

    r0 = s * _RPT
    nz = jnp.where(s < _NS - 1, _RPT // _ZB, _RPT_LAST // _ZB)

    def zcopy(i, _):
        off = pl.multiple_of(r0 + i * _ZB, 8)
        pltpu.sync_copy(rows1.at[pl.ds(0, _ZB)], acc_sh.at[pl.ds(off, _ZB)])
        return 0
    lax.fori_loop(0, nz, zcopy, 0)

    for b in range(_NBUF):
        pltpu.async_copy(x2_hbm.at[src_v.at[b]], rows[b], sems[b])

    plsc.subcore_barrier()

    def process(kb, b):
        pltpu.make_async_copy(x2_hbm.at[src_v.at[kb]], rows[b], sems[b]).wait()

        @plsc.parallel_loop(0, _C // _L, unroll=4)
        def mul(g):
            wv = w_v[kb, pl.ds(g * _L, _L)]
            for l in range(_L):
                w = wv[l]
                e = g * _L + l
                for j in range(_DH // _L):
                    sl = pl.ds(j * _L, _L)
                    rows[b][e, sl] = rows[b][e, sl] * w

        pltpu.sync_copy(rows[b], acc_sh.at[dst_v.at[kb]], add=True)

    def body(i, _):
        k = i * _NBUF
        for b in range(_NBUF):
            kb = k + b
            process(kb, b)

            @pl.when(kb + _NBUF <= _NCHUNK - 1)
            def _():
                pltpu.async_copy(x2_hbm.at[src_v.at[kb + _NBUF]],
                                 rows[b], sems[b])
        return 0
    lax.fori_loop(0, (_NCHUNK - 1) // _NBUF, body, 0)

    process(_NCHUNK - 1, 0)

    plsc.subcore_barrier()
    ro = pl.multiple_of(r0, 8)

    @pl.when(s < _NS - 1)
    def _():
        pltpu.sync_copy(acc_sh.at[pl.ds(ro, _RPT)],
                        out_hbm.at[pl.ds(ro, _RPT), c])

    @pl.when(s == _NS - 1)
    def _():
        pltpu.sync_copy(acc_sh.at[pl.ds(ro, _RPT_LAST)],
                        out_hbm.at[pl.ds(ro, _RPT_LAST), c])


def _spmm(x2, src4, dst3, w3):
    mesh = plsc.VectorSubcoreMesh(core_axis_name="c", subcore_axis_name="s")
    f = pl.kernel(
        _spmm_body,
        mesh=mesh,
        compiler_params=pltpu.CompilerParams(use_tc_tiling_on_sc=False),
        out_type=jax.ShapeDtypeStruct((_N, _NC, _DH), jnp.float32),
        scratch_types=[
            pltpu.VMEM((_NCHUNK, _C), jnp.int32),
            pltpu.VMEM((_NCHUNK, _C), jnp.int32),
            pltpu.VMEM((_NCHUNK, _C), jnp.float32),
            pltpu.VMEM((_C, _DH), jnp.float32),
            pltpu.VMEM((_C, _DH), jnp.float32),
            pltpu.VMEM((_C, _DH), jnp.float32),
            pltpu.VMEM_SHARED((_N, _DH), jnp.float32),
            pltpu.SemaphoreType.DMA,
            pltpu.SemaphoreType.DMA,
            pltpu.SemaphoreType.DMA,
        ],
    )
    return f(x2, src4, dst3, w3)


_R = 2000


def _tc1_body(x_ref, p_ref, wa_ref, ba_ref, wla_ref, wlb_ref, bl_ref, h_ref):
    agg = jnp.maximum(
        jnp.dot(p_ref[...], wa_ref[...], preferred_element_type=jnp.float32)
        + ba_ref[...], 0.0)
    hv = (jnp.dot(x_ref[...], wla_ref[...], preferred_element_type=jnp.float32)
          + jnp.dot(agg, wlb_ref[...], preferred_element_type=jnp.float32)
          + bl_ref[...])
    h_ref[...] = jnp.maximum(hv, 0.0)


def _tc2_body(h_ref, q_ref, wa_ref, ba_ref, wla_ref, wlb_ref, bl_ref, o_ref):
    agg = jnp.maximum(
        jnp.dot(q_ref[...], wa_ref[...], preferred_element_type=jnp.float32)
        + ba_ref[...], 0.0)
    ov = (jnp.dot(h_ref[...], wla_ref[...], preferred_element_type=jnp.float32)
          + jnp.dot(agg, wlb_ref[...], preferred_element_type=jnp.float32)
          + bl_ref[...])
    nrm = jnp.sqrt(jnp.sum(ov * ov, axis=1, keepdims=True))
    o_ref[...] = ov / jnp.maximum(nrm, 1e-12)


def _dense_layer(body, xh, p, Wa, ba, Wl, bl):
    grid = (_N // _R,)
    specs = [
        pl.BlockSpec((_R, _D), lambda i: (i, 0)),
        pl.BlockSpec((_R, _D), lambda i: (i, 0)),
        pl.BlockSpec((_D, _D), lambda i: (0, 0)),
        pl.BlockSpec((1, _D), lambda i: (0, 0)),
        pl.BlockSpec((_D, _D), lambda i: (0, 0)),
        pl.BlockSpec((_D, _D), lambda i: (0, 0)),
        pl.BlockSpec((1, _D), lambda i: (0, 0)),
    ]
    return pl.pallas_call(
        body,
        grid=grid,
        in_specs=specs,
        out_specs=pl.BlockSpec((_R, _D), lambda i: (i, 0)),
        out_shape=jax.ShapeDtypeStruct((_N, _D), jnp.float32),
    )(xh, p, Wa, ba.reshape(1, _D), Wl[:_D], Wl[_D:], bl.reshape(1, _D))


def kernel(x, edge_index, edge_weight, Wa0, ba0, Wa1, ba1, Wl0, bl0, Wl1, bl1):
    npad = _EPAD - _E
    src = edge_index[1].astype(jnp.int32)
    dst = jnp.pad(edge_index[0].astype(jnp.int32), (0, npad))
    w = jnp.pad(edge_weight.astype(jnp.float32), (0, npad))

    src2 = src * 2
    src4 = jnp.stack([src2, src2 + 1])
    src4 = jnp.pad(src4, ((0, 0), (0, npad)))
    src4 = src4.reshape(_NC, _NS, _NCHUNK, _C)
    dst3 = dst.reshape(_NS, _NCHUNK, _C)
    w3 = w.reshape(_NS, _NCHUNK, _C)

    x2 = x.reshape(2 * _N, _DH)
    p = _spmm(x2, src4, dst3, w3).reshape(_N, _D)
    h = _dense_layer(_tc1_body, x, p, Wa0, ba0, Wl0, bl0)
    q = _spmm(h.reshape(2 * _N, _DH), src4, dst3, w3).reshape(_N, _D)
    return _dense_layer(_tc2_body, h, q, Wa1, ba1, Wl1, bl1)


# speedup vs baseline: 1.0985x; 1.0043x over previous
import jax
import jax.numpy as jnp
from jax import lax
from jax.experimental import pallas as pl
from jax.experimental.pallas import tpu as pltpu
from jax.experimental.pallas import tpu_sc as plsc

_N = 10000
_D = 128
_DH = _D // 2
_NC = 2
_NS = 16
_L = 16

_C = 128
_E = 320000
_EPAD = ((_E + _NS * _C - 1) // (_NS * _C)) * (_NS * _C)
_NCHUNK = _EPAD // (_NS * _C)
_NBUF = 3

_RPT = 640
_RPT_LAST = _N - 15 * _RPT
_ZB = 80


def _spmm_body(x2_hbm, src_hbm, dst_hbm, w_hbm, out_hbm,
               src_v, dst_v, w_v,
               rows0, rows1, rows2, acc_sh,
               sem0, sem1, sem2):
    c = lax.axis_index("c")
    s = lax.axis_index("s")
    rows = (rows0, rows1, rows2)
    sems = (sem0, sem1, sem2)

    pltpu.sync_copy(src_hbm.at[c, s], src_v)
    pltpu.sync_copy(dst_hbm.at[s], dst_v)
    pltpu.sync_copy(w_hbm.at[s], w_v)

    zero16 = jnp.zeros((_L,), jnp.float32)

    def zrow(i, _):
        for j in range(_DH // _L):
            rows1[i, pl.ds(j * _L, _L)] = zero16
        return 0
    lax.fori_loop(0, _ZB, zrow, 0)

    r0 = s * _RPT
    nz = jnp.where(s < _NS - 1, _RPT // _ZB, _RPT_LAST // _ZB)

    def zcopy(i, _):
        off = pl.multiple_of(r0 + i * _ZB, 8)
        pltpu.sync_copy(rows1.at[pl.ds(0, _ZB)], acc_sh.at[pl.ds(off, _ZB)])
        return 0
    lax.fori_loop(0, nz, zcopy, 0)

    for b in range(_NBUF):
        pltpu.async_copy(x2_hbm.at[src_v.at[b]], rows[b], sems[b])

    plsc.subcore_barrier()

    def process(kb, b):
        pltpu.make_async_copy(x2_hbm.at[src_v.at[kb]], rows[b], sems[b]).wait()

        @plsc.parallel_loop(0, _C // _L, unroll=4)
        def mul(g):
            wv = w_v[kb, pl.ds(g * _L, _L)]
            for l in range(_L):
                w = wv[l]
                e = g * _L + l
                for j in range(_DH // _L):
                    sl = pl.ds(j * _L, _L)
                    rows[b][e, sl] = rows[b][e, sl] * w

        pltpu.sync_copy(rows[b], acc_sh.at[dst_v.at[kb]], add=True)

    def body(i, _):
        k = i * _NBUF
        for b in range(_NBUF):
            kb = k + b
            process(kb, b)

            @pl.when(kb + _NBUF <= _NCHUNK - 1)
            def _():
                pltpu.async_copy(x2_hbm.at[src_v.at[kb + _NBUF]],
                                 rows[b], sems[b])
        return 0
    lax.fori_loop(0, (_NCHUNK - 1) // _NBUF, body, 0)

    process(_NCHUNK - 1, 0)

    plsc.subcore_barrier()
    ro = pl.multiple_of(r0, 8)

    @pl.when(s < _NS - 1)
    def _():
        pltpu.sync_copy(acc_sh.at[pl.ds(ro, _RPT)],
                        out_hbm.at[pl.ds(ro, _RPT), c])

    @pl.when(s == _NS - 1)
    def _():
        pltpu.sync_copy(acc_sh.at[pl.ds(ro, _RPT_LAST)],
                        out_hbm.at[pl.ds(ro, _RPT_LAST), c])


def _spmm(x2, src4, dst3, w3):
    mesh = plsc.VectorSubcoreMesh(core_axis_name="c", subcore_axis_name="s")
    f = pl.kernel(
        _spmm_body,
        mesh=mesh,
        compiler_params=pltpu.CompilerParams(use_tc_tiling_on_sc=False),
        out_type=jax.ShapeDtypeStruct((_N, _NC, _DH), jnp.float32),
        scratch_types=[
            pltpu.VMEM((_NCHUNK, _C), jnp.int32),
            pltpu.VMEM((_NCHUNK, _C), jnp.int32),
            pltpu.VMEM((_NCHUNK, _C), jnp.float32),
            pltpu.VMEM((_C, _DH), jnp.float32),
            pltpu.VMEM((_C, _DH), jnp.float32),
            pltpu.VMEM((_C, _DH), jnp.float32),
            pltpu.VMEM_SHARED((_N, _DH), jnp.float32),
            pltpu.SemaphoreType.DMA,
            pltpu.SemaphoreType.DMA,
            pltpu.SemaphoreType.DMA,
        ],
    )
    return f(x2, src4, dst3, w3)


_R = 5000


def _tc1_body(x_ref, p_ref, wa_ref, ba_ref, wla_ref, wlb_ref, bl_ref, h_ref):
    agg = jnp.maximum(
        jnp.dot(p_ref[...], wa_ref[...], preferred_element_type=jnp.float32)
        + ba_ref[...], 0.0)
    hv = (jnp.dot(x_ref[...], wla_ref[...], preferred_element_type=jnp.float32)
          + jnp.dot(agg, wlb_ref[...], preferred_element_type=jnp.float32)
          + bl_ref[...])
    h_ref[...] = jnp.maximum(hv, 0.0)


def _tc2_body(h_ref, q_ref, wa_ref, ba_ref, wla_ref, wlb_ref, bl_ref, o_ref):
    agg = jnp.maximum(
        jnp.dot(q_ref[...], wa_ref[...], preferred_element_type=jnp.float32)
        + ba_ref[...], 0.0)
    ov = (jnp.dot(h_ref[...], wla_ref[...], preferred_element_type=jnp.float32)
          + jnp.dot(agg, wlb_ref[...], preferred_element_type=jnp.float32)
          + bl_ref[...])
    nrm = jnp.sqrt(jnp.sum(ov * ov, axis=1, keepdims=True))
    o_ref[...] = ov / jnp.maximum(nrm, 1e-12)


def _dense_layer(body, xh, p, Wa, ba, Wl, bl):
    grid = (_N // _R,)
    specs = [
        pl.BlockSpec((_R, _D), lambda i: (i, 0)),
        pl.BlockSpec((_R, _D), lambda i: (i, 0)),
        pl.BlockSpec((_D, _D), lambda i: (0, 0)),
        pl.BlockSpec((1, _D), lambda i: (0, 0)),
        pl.BlockSpec((_D, _D), lambda i: (0, 0)),
        pl.BlockSpec((_D, _D), lambda i: (0, 0)),
        pl.BlockSpec((1, _D), lambda i: (0, 0)),
    ]
    return pl.pallas_call(
        body,
        grid=grid,
        in_specs=specs,
        out_specs=pl.BlockSpec((_R, _D), lambda i: (i, 0)),
        out_shape=jax.ShapeDtypeStruct((_N, _D), jnp.float32),
    )(xh, p, Wa, ba.reshape(1, _D), Wl[:_D], Wl[_D:], bl.reshape(1, _D))


def kernel(x, edge_index, edge_weight, Wa0, ba0, Wa1, ba1, Wl0, bl0, Wl1, bl1):
    npad = _EPAD - _E
    src = edge_index[1].astype(jnp.int32)
    dst = jnp.pad(edge_index[0].astype(jnp.int32), (0, npad))
    w = jnp.pad(edge_weight.astype(jnp.float32), (0, npad))

    src2 = src * 2
    src4 = jnp.stack([src2, src2 + 1])
    src4 = jnp.pad(src4, ((0, 0), (0, npad)))
    src4 = src4.reshape(_NC, _NS, _NCHUNK, _C)
    dst3 = dst.reshape(_NS, _NCHUNK, _C)
    w3 = w.reshape(_NS, _NCHUNK, _C)

    x2 = x.reshape(2 * _N, _DH)
    p = _spmm(x2, src4, dst3, w3).reshape(_N, _D)
    h = _dense_layer(_tc1_body, x, p, Wa0, ba0, Wl0, bl0)
    q = _spmm(h.reshape(2 * _N, _DH), src4, dst3, w3).reshape(_N, _D)
    return _dense_layer(_tc2_body, h, q, Wa1, ba1, Wl1, bl1)


# speedup vs baseline: 1.1008x; 1.0021x over previous
import jax
import jax.numpy as jnp
from jax import lax
from jax.experimental import pallas as pl
from jax.experimental.pallas import tpu as pltpu
from jax.experimental.pallas import tpu_sc as plsc

_N = 10000
_D = 128
_DH = _D // 2
_NC = 2
_NS = 16
_L = 16

_C = 128
_E = 320000
_EPAD = ((_E + _NS * _C - 1) // (_NS * _C)) * (_NS * _C)
_NCHUNK = _EPAD // (_NS * _C)
_NBUF = 3

_RPT = 640
_RPT_LAST = _N - 15 * _RPT
_ZB = 80


def _spmm_body(x2_hbm, src_hbm, dst_hbm, w_hbm, out_hbm,
               src_v, dst_v, w_v,
               rows0, rows1, rows2, acc_sh,
               sem0, sem1, sem2):
    c = lax.axis_index("c")
    s = lax.axis_index("s")
    rows = (rows0, rows1, rows2)
    sems = (sem0, sem1, sem2)

    pltpu.sync_copy(src_hbm.at[c, s], src_v)
    pltpu.sync_copy(dst_hbm.at[s], dst_v)
    pltpu.sync_copy(w_hbm.at[s], w_v)

    zero16 = jnp.zeros((_L,), jnp.float32)

    def zrow(i, _):
        for j in range(_DH // _L):
            rows1[i, pl.ds(j * _L, _L)] = zero16
        return 0
    lax.fori_loop(0, _ZB, zrow, 0)

    r0 = s * _RPT
    nz = jnp.where(s < _NS - 1, _RPT // _ZB, _RPT_LAST // _ZB)

    def zcopy(i, _):
        off = pl.multiple_of(r0 + i * _ZB, 8)
        pltpu.sync_copy(rows1.at[pl.ds(0, _ZB)], acc_sh.at[pl.ds(off, _ZB)])
        return 0
    lax.fori_loop(0, nz, zcopy, 0)

    for b in range(_NBUF):
        pltpu.async_copy(x2_hbm.at[src_v.at[b]], rows[b], sems[b])

    plsc.subcore_barrier()

    def process(kb, b):
        pltpu.make_async_copy(x2_hbm.at[src_v.at[kb]], rows[b], sems[b]).wait()

        @plsc.parallel_loop(0, _C // _L, unroll=2)
        def mul(g):
            wv = w_v[kb, pl.ds(g * _L, _L)]
            for l in range(_L):
                w = wv[l]
                e = g * _L + l
                for j in range(_DH // _L):
                    sl = pl.ds(j * _L, _L)
                    rows[b][e, sl] = rows[b][e, sl] * w

        pltpu.sync_copy(rows[b], acc_sh.at[dst_v.at[kb]], add=True)

    def body(i, _):
        k = i * _NBUF
        for b in range(_NBUF):
            kb = k + b
            process(kb, b)

            @pl.when(kb + _NBUF <= _NCHUNK - 1)
            def _():
                pltpu.async_copy(x2_hbm.at[src_v.at[kb + _NBUF]],
                                 rows[b], sems[b])
        return 0
    lax.fori_loop(0, (_NCHUNK - 1) // _NBUF, body, 0)

    process(_NCHUNK - 1, 0)

    plsc.subcore_barrier()
    ro = pl.multiple_of(r0, 8)

    @pl.when(s < _NS - 1)
    def _():
        pltpu.sync_copy(acc_sh.at[pl.ds(ro, _RPT)],
                        out_hbm.at[pl.ds(ro, _RPT), c])

    @pl.when(s == _NS - 1)
    def _():
        pltpu.sync_copy(acc_sh.at[pl.ds(ro, _RPT_LAST)],
                        out_hbm.at[pl.ds(ro, _RPT_LAST), c])


def _spmm(x2, src4, dst3, w3):
    mesh = plsc.VectorSubcoreMesh(core_axis_name="c", subcore_axis_name="s")
    f = pl.kernel(
        _spmm_body,
        mesh=mesh,
        compiler_params=pltpu.CompilerParams(use_tc_tiling_on_sc=False),
        out_type=jax.ShapeDtypeStruct((_N, _NC, _DH), jnp.float32),
        scratch_types=[
            pltpu.VMEM((_NCHUNK, _C), jnp.int32),
            pltpu.VMEM((_NCHUNK, _C), jnp.int32),
            pltpu.VMEM((_NCHUNK, _C), jnp.float32),
            pltpu.VMEM((_C, _DH), jnp.float32),
            pltpu.VMEM((_C, _DH), jnp.float32),
            pltpu.VMEM((_C, _DH), jnp.float32),
            pltpu.VMEM_SHARED((_N, _DH), jnp.float32),
            pltpu.SemaphoreType.DMA,
            pltpu.SemaphoreType.DMA,
            pltpu.SemaphoreType.DMA,
        ],
    )
    return f(x2, src4, dst3, w3)


_R = 5000


def _tc1_body(x_ref, p_ref, wa_ref, ba_ref, wla_ref, wlb_ref, bl_ref, h_ref):
    agg = jnp.maximum(
        jnp.dot(p_ref[...], wa_ref[...], preferred_element_type=jnp.float32)
        + ba_ref[...], 0.0)
    hv = (jnp.dot(x_ref[...], wla_ref[...], preferred_element_type=jnp.float32)
          + jnp.dot(agg, wlb_ref[...], preferred_element_type=jnp.float32)
          + bl_ref[...])
    h_ref[...] = jnp.maximum(hv, 0.0)


def _tc2_body(h_ref, q_ref, wa_ref, ba_ref, wla_ref, wlb_ref, bl_ref, o_ref):
    agg = jnp.maximum(
        jnp.dot(q_ref[...], wa_ref[...], preferred_element_type=jnp.float32)
        + ba_ref[...], 0.0)
    ov = (jnp.dot(h_ref[...], wla_ref[...], preferred_element_type=jnp.float32)
          + jnp.dot(agg, wlb_ref[...], preferred_element_type=jnp.float32)
          + bl_ref[...])
    nrm = jnp.sqrt(jnp.sum(ov * ov, axis=1, keepdims=True))
    o_ref[...] = ov / jnp.maximum(nrm, 1e-12)


def _dense_layer(body, xh, p, Wa, ba, Wl, bl):
    grid = (_N // _R,)
    specs = [
        pl.BlockSpec((_R, _D), lambda i: (i, 0)),
        pl.BlockSpec((_R, _D), lambda i: (i, 0)),
        pl.BlockSpec((_D, _D), lambda i: (0, 0)),
        pl.BlockSpec((1, _D), lambda i: (0, 0)),
        pl.BlockSpec((_D, _D), lambda i: (0, 0)),
        pl.BlockSpec((_D, _D), lambda i: (0, 0)),
        pl.BlockSpec((1, _D), lambda i: (0, 0)),
    ]
    return pl.pallas_call(
        body,
        grid=grid,
        in_specs=specs,
        out_specs=pl.BlockSpec((_R, _D), lambda i: (i, 0)),
        out_shape=jax.ShapeDtypeStruct((_N, _D), jnp.float32),
    )(xh, p, Wa, ba.reshape(1, _D), Wl[:_D], Wl[_D:], bl.reshape(1, _D))


def kernel(x, edge_index, edge_weight, Wa0, ba0, Wa1, ba1, Wl0, bl0, Wl1, bl1):
    npad = _EPAD - _E
    src = edge_index[1].astype(jnp.int32)
    dst = jnp.pad(edge_index[0].astype(jnp.int32), (0, npad))
    w = jnp.pad(edge_weight.astype(jnp.float32), (0, npad))

    src2 = src * 2
    src4 = jnp.stack([src2, src2 + 1])
    src4 = jnp.pad(src4, ((0, 0), (0, npad)))
    src4 = src4.reshape(_NC, _NS, _NCHUNK, _C)
    dst3 = dst.reshape(_NS, _NCHUNK, _C)
    w3 = w.reshape(_NS, _NCHUNK, _C)

    x2 = x.reshape(2 * _N, _DH)
    p = _spmm(x2, src4, dst3, w3).reshape(_N, _D)
    h = _dense_layer(_tc1_body, x, p, Wa0, ba0, Wl0, bl0)
    q = _spmm(h.reshape(2 * _N, _DH), src4, dst3, w3).reshape(_N, _D)
    return _dense_layer(_tc2_body, h, q, Wa1, ba1, Wl1, bl1)
